# baseline (speedup 1.0000x reference)
import jax
import jax.numpy as jnp
from jax import lax
from jax.experimental import pallas as pl
from jax.experimental.pallas import tpu as pltpu
from jax.experimental.pallas import tpu_sc as plsc

B, N, D = 8, 1250, 128
E = 40000
L = 4
Np = 1280
NC, NS = 2, 16
EP = E // NS
ROWS = 20
EPP = ROWS * 128
NPNP = Np * Np
STRIPE = NPNP // NS
ZCH = STRIPE // 8
GPC = B // NC
CB = Np // 128
CBSZ = Np * 128
TG = 2


def _sc_body(dst_hbm, src_hbm, out_hbm,
             shared, srcv, dstv, idxb, valb, nvalb, zbuf,
             sem1, sem2, sem3):
  c = lax.axis_index("c")
  s = lax.axis_index("s")

  for i in range(8):
    valb[pl.ds(i * 16, 16)] = jnp.ones((16,), jnp.float32)
    nvalb[pl.ds(i * 16, 16)] = -jnp.ones((16,), jnp.float32)

  def _z(i, _):
    zbuf[pl.ds(i * 16, 16)] = jnp.zeros((16,), jnp.float32)
    return 0
  lax.fori_loop(0, ZCH // 16, _z, 0)
  soff = pl.multiple_of(s * STRIPE, 256)
  for q in range(8):
    pltpu.sync_copy(zbuf, shared.at[pl.ds(soff + q * ZCH, ZCH)])
  plsc.subcore_barrier()

  fetches = [None, None]

  def _fetch(b):
    return (pltpu.async_copy(src_hbm.at[b, s], srcv, sem1),
            pltpu.async_copy(dst_hbm.at[b, s], dstv, sem2))

  b0 = c * GPC
  fetches = _fetch(b0)
  for r in range(GPC):
    b = b0 + r
    fetches[0].wait()
    fetches[1].wait()
    adds = []
    for j in range(ROWS):
      for k in range(8):
        sl = pl.ds(k * 16, 16)
        sv = srcv[j, sl]
        dv = dstv[j, sl]
        idxb[j, sl] = (lax.shift_right_logical(sv, 7) * CBSZ
                       + dv * 128 + lax.bitwise_and(sv, 127))
      adds.append(pltpu.async_copy(valb, shared.at[idxb.at[j]],
                                   sem3, add=True))
    for cp in adds:
      cp.wait()
    plsc.subcore_barrier()
    obase = pl.multiple_of(b * NPNP, 256)
    out_cp = pltpu.async_copy(shared.at[pl.ds(soff, STRIPE)],
                              out_hbm.at[pl.ds(obase + soff, STRIPE)], sem1)
    out_cp.wait()
    plsc.subcore_barrier()
    if r + 1 < GPC:
      fetches = _fetch(b + 1)
      subs = [pltpu.async_copy(nvalb, shared.at[idxb.at[j]],
                               sem3, add=True) for j in range(ROWS)]
      for cp in subs:
        cp.wait()
      plsc.subcore_barrier()


def _build_adjacency(dst4, src4):
  mesh = plsc.VectorSubcoreMesh(core_axis_name="c", subcore_axis_name="s")
  f = pl.kernel(
      _sc_body,
      out_type=jax.ShapeDtypeStruct((B * NPNP,), jnp.float32),
      mesh=mesh,
      scratch_types=[
          pltpu.VMEM_SHARED((NPNP + 256,), jnp.float32),
          pltpu.VMEM((ROWS, 128), jnp.int32),
          pltpu.VMEM((ROWS, 128), jnp.int32),
          pltpu.VMEM((ROWS, 128), jnp.int32),
          pltpu.VMEM((128,), jnp.float32),
          pltpu.VMEM((128,), jnp.float32),
          pltpu.VMEM((ZCH,), jnp.float32),
          pltpu.SemaphoreType.DMA,
          pltpu.SemaphoreType.DMA,
          pltpu.SemaphoreType.DMA,
      ],
  )
  return f(dst4, src4)


def _tc_body(a_hbm, x_ref, w_ref, b_ref, o_ref,
             an0_ref, an1_ref, slab0, slab1, sem0, sem1):
  step = pl.program_id(0)
  ans = (an0_ref, an1_ref)
  slabs = (slab0, slab1)
  sems = (sem0, sem1)
  seq = [(g, cb) for g in range(TG) for cb in range(CB)]
  copies = [None, None]
  copies[0] = pltpu.async_copy(a_hbm.at[step * TG, 0], slabs[0], sems[0])
  for i, (g, cb) in enumerate(seq):
    if i + 1 < len(seq):
      g2, cb2 = seq[i + 1]
      copies[(i + 1) % 2] = pltpu.async_copy(
          a_hbm.at[step * TG + g2, cb2], slabs[(i + 1) % 2], sems[(i + 1) % 2])
    copies[i % 2].wait()
    ans[g][:, 128 * cb:128 * (cb + 1)] = slabs[i % 2][...].astype(jnp.bfloat16)
  ones = jnp.ones((Np, 1), jnp.bfloat16)
  dis = []
  for g in range(TG):
    deg = jnp.dot(ans[g][...], ones, preferred_element_type=jnp.float32) + 1.0
    dis.append(lax.rsqrt(deg))
  xs = [x_ref[g] for g in range(TG)]
  for l in range(L):
    w = w_ref[l]
    bias = b_ref[l][None, :]
    hs = [None] * TG
    for g in range(TG):
      h = jnp.dot(xs[g], w, preferred_element_type=jnp.float32)
      hs[g] = h * dis[g]
    for g in range(TG):
      y = jnp.dot(ans[g][...], hs[g].astype(jnp.bfloat16),
                  preferred_element_type=jnp.float32) + hs[g]
      xs[g] = jnp.tanh(y * dis[g] + bias)
  for g in range(TG):
    o_ref[g] = xs[g]


def _gcn_stack(adj, x_pad, wst, bst):
  return pl.pallas_call(
      _tc_body,
      grid=(B // TG,),
      in_specs=[
          pl.BlockSpec(memory_space=pltpu.HBM),
          pl.BlockSpec((TG, Np, D), lambda b: (b, 0, 0)),
          pl.BlockSpec((L, D, D), lambda b: (0, 0, 0)),
          pl.BlockSpec((L, D), lambda b: (0, 0)),
      ],
      out_specs=pl.BlockSpec((TG, Np, D), lambda b: (b, 0, 0)),
      out_shape=jax.ShapeDtypeStruct((B, N, D), jnp.float32),
      scratch_shapes=[pltpu.VMEM((Np, Np), jnp.bfloat16),
                      pltpu.VMEM((Np, Np), jnp.bfloat16),
                      pltpu.VMEM((Np, 128), jnp.float32),
                      pltpu.VMEM((Np, 128), jnp.float32),
                      pltpu.SemaphoreType.DMA,
                      pltpu.SemaphoreType.DMA],
  )(adj, x_pad, wst, bst)


@jax.jit
def kernel(batch_node_tsr, edge_tsr_list, batch_last_node_idx_list,
           W0, b0, W1, b1, W2, b2, W3, b3):
  del batch_last_node_idx_list
  src = edge_tsr_list[:, 0, :].reshape(B, NS, EP)
  dst = edge_tsr_list[:, 1, :].reshape(B, NS, EP)
  pad = ((0, 0), (0, 0), (0, EPP - EP))
  src4 = jnp.pad(src, pad, constant_values=Np).reshape(B, NS, ROWS, 128)
  dst4 = jnp.pad(dst, pad).reshape(B, NS, ROWS, 128)
  adj = _build_adjacency(dst4, src4).reshape(B, CB, Np, 128)

  x_pad = jnp.pad(batch_node_tsr, ((0, 0), (0, Np - N), (0, 0)))
  wst = jnp.stack([W0, W1, W2, W3])
  bst = jnp.stack([b0, b1, b2, b3])
  return _gcn_stack(adj, x_pad, wst, bst)


# speedup vs baseline: 1.2510x; 1.2510x over previous
import jax
import jax.numpy as jnp
from jax import lax
from jax.experimental import pallas as pl
from jax.experimental.pallas import tpu as pltpu
from jax.experimental.pallas import tpu_sc as plsc

B, N, D = 8, 1250, 128
E = 40000
L = 4
Np = 1280
NC, NS = 2, 16
EP = E // NS
ROWS = 20
EPP = ROWS * 128
NPNP = Np * Np
STRIPE = NPNP // NS
ZCH = STRIPE // 8
GPC = B // NC
CB = Np // 128
CBSZ = Np * 128
TG = 2


def _sc_body(dst_hbm, src_hbm, out_hbm,
             shared, srcv, dstv, idxb, valb, nvalb, zbuf,
             sem1, sem2, sem3):
  c = lax.axis_index("c")
  s = lax.axis_index("s")

  for i in range(8):
    valb[pl.ds(i * 16, 16)] = jnp.ones((16,), jnp.float32)
    nvalb[pl.ds(i * 16, 16)] = -jnp.ones((16,), jnp.float32)

  def _z(i, _):
    zbuf[pl.ds(i * 16, 16)] = jnp.zeros((16,), jnp.float32)
    return 0
  lax.fori_loop(0, ZCH // 16, _z, 0)
  soff = pl.multiple_of(s * STRIPE, 256)
  for q in range(8):
    pltpu.sync_copy(zbuf, shared.at[pl.ds(soff + q * ZCH, ZCH)])
  plsc.subcore_barrier()

  fetches = [None, None]

  def _fetch(b):
    return (pltpu.async_copy(src_hbm.at[b, s], srcv, sem1),
            pltpu.async_copy(dst_hbm.at[b, s], dstv, sem2))

  b0 = c * GPC
  fetches = _fetch(b0)
  for r in range(GPC):
    b = b0 + r
    fetches[0].wait()
    fetches[1].wait()
    adds = []
    for j in range(ROWS):
      for k in range(8):
        sl = pl.ds(k * 16, 16)
        sv = srcv[j, sl]
        dv = dstv[j, sl]
        idxb[j, sl] = (lax.shift_right_logical(sv, 7) * CBSZ
                       + dv * 128 + lax.bitwise_and(sv, 127))
      adds.append(pltpu.async_copy(valb, shared.at[idxb.at[j]],
                                   sem3, add=True))
    for cp in adds:
      cp.wait()
    plsc.subcore_barrier()
    obase = pl.multiple_of(b * NPNP, 256)
    out_cp = pltpu.async_copy(shared.at[pl.ds(soff, STRIPE)],
                              out_hbm.at[pl.ds(obase + soff, STRIPE)], sem1)
    out_cp.wait()
    plsc.subcore_barrier()
    if r + 1 < GPC:
      fetches = _fetch(b + 1)
      subs = [pltpu.async_copy(nvalb, shared.at[idxb.at[j]],
                               sem3, add=True) for j in range(ROWS)]
      for cp in subs:
        cp.wait()
      plsc.subcore_barrier()


def _build_adjacency(dst4, src4):
  mesh = plsc.VectorSubcoreMesh(core_axis_name="c", subcore_axis_name="s")
  f = pl.kernel(
      _sc_body,
      out_type=jax.ShapeDtypeStruct((B * NPNP,), jnp.float32),
      mesh=mesh,
      scratch_types=[
          pltpu.VMEM_SHARED((NPNP + 256,), jnp.float32),
          pltpu.VMEM((ROWS, 128), jnp.int32),
          pltpu.VMEM((ROWS, 128), jnp.int32),
          pltpu.VMEM((ROWS, 128), jnp.int32),
          pltpu.VMEM((128,), jnp.float32),
          pltpu.VMEM((128,), jnp.float32),
          pltpu.VMEM((ZCH,), jnp.float32),
          pltpu.SemaphoreType.DMA,
          pltpu.SemaphoreType.DMA,
          pltpu.SemaphoreType.DMA,
      ],
  )
  return f(dst4, src4)


def _tc_body(a_ref, x_ref, w_ref, b_ref, o_ref, an0_ref, an1_ref):
  h = pl.program_id(1)
  ans = (an0_ref, an1_ref)
  HCB = CB // 2

  def _asm(base):
    for g in range(TG):
      for cbl in range(HCB):
        cb = base + cbl
        ans[g][:, 128 * cb:128 * (cb + 1)] = a_ref[g, cbl].astype(jnp.bfloat16)

  @pl.when(h == 0)
  def _():
    _asm(0)

  @pl.when(h == 1)
  def _():
    _asm(HCB)
    ones = jnp.ones((Np, 1), jnp.bfloat16)
    dis = []
    for g in range(TG):
      deg = jnp.dot(ans[g][...], ones,
                    preferred_element_type=jnp.float32) + 1.0
      dis.append(lax.rsqrt(deg))
    xs = [x_ref[g] for g in range(TG)]
    for l in range(L):
      w = w_ref[l]
      bias = b_ref[l][None, :]
      hs = [None] * TG
      for g in range(TG):
        hh = jnp.dot(xs[g], w, preferred_element_type=jnp.float32)
        hs[g] = hh * dis[g]
      for g in range(TG):
        y = jnp.dot(ans[g][...], hs[g].astype(jnp.bfloat16),
                    preferred_element_type=jnp.float32) + hs[g]
        xs[g] = jnp.tanh(y * dis[g] + bias)
    for g in range(TG):
      o_ref[g] = xs[g]


def _gcn_stack(adj, x_pad, wst, bst):
  return pl.pallas_call(
      _tc_body,
      grid=(B // TG, 2),
      in_specs=[
          pl.BlockSpec((TG, CB // 2, Np, 128), lambda b, h: (b, h, 0, 0)),
          pl.BlockSpec((TG, Np, D), lambda b, h: (b, 0, 0)),
          pl.BlockSpec((L, D, D), lambda b, h: (0, 0, 0)),
          pl.BlockSpec((L, D), lambda b, h: (0, 0)),
      ],
      out_specs=pl.BlockSpec((TG, Np, D), lambda b, h: (b, 0, 0)),
      out_shape=jax.ShapeDtypeStruct((B, N, D), jnp.float32),
      scratch_shapes=[pltpu.VMEM((Np, Np), jnp.bfloat16),
                      pltpu.VMEM((Np, Np), jnp.bfloat16)],
  )(adj, x_pad, wst, bst)


@jax.jit
def kernel(batch_node_tsr, edge_tsr_list, batch_last_node_idx_list,
           W0, b0, W1, b1, W2, b2, W3, b3):
  del batch_last_node_idx_list
  src = edge_tsr_list[:, 0, :].reshape(B, NS, EP)
  dst = edge_tsr_list[:, 1, :].reshape(B, NS, EP)
  pad = ((0, 0), (0, 0), (0, EPP - EP))
  src4 = jnp.pad(src, pad, constant_values=Np).reshape(B, NS, ROWS, 128)
  dst4 = jnp.pad(dst, pad).reshape(B, NS, ROWS, 128)
  adj = _build_adjacency(dst4, src4).reshape(B, CB, Np, 128)

  x_pad = jnp.pad(batch_node_tsr, ((0, 0), (0, Np - N), (0, 0)))
  wst = jnp.stack([W0, W1, W2, W3])
  bst = jnp.stack([b0, b1, b2, b3])
  return _gcn_stack(adj, x_pad, wst, bst)


# speedup vs baseline: 1.2528x; 1.0014x over previous
import jax
import jax.numpy as jnp
from jax import lax
from jax.experimental import pallas as pl
from jax.experimental.pallas import tpu as pltpu
from jax.experimental.pallas import tpu_sc as plsc

B, N, D = 8, 1250, 128
E = 40000
L = 4
Np = 1280
NC, NS = 2, 16
EP = E // NS
ROWS = 20
EPP = ROWS * 128
NPNP = Np * Np
STRIPE = NPNP // NS
ZCH = STRIPE // 8
GPC = B // NC
CB = Np // 128
CBSZ = Np * 128
TG = 2
HSTEPS = 2


def _sc_body(dst_hbm, src_hbm, out_hbm,
             shared, srcv, dstv, idxb, valb, nvalb, zbuf,
             sem1, sem2, sem3):
  c = lax.axis_index("c")
  s = lax.axis_index("s")

  for i in range(8):
    valb[pl.ds(i * 16, 16)] = jnp.ones((16,), jnp.float32)
    nvalb[pl.ds(i * 16, 16)] = -jnp.ones((16,), jnp.float32)

  def _z(i, _):
    zbuf[pl.ds(i * 16, 16)] = jnp.zeros((16,), jnp.float32)
    return 0
  lax.fori_loop(0, ZCH // 16, _z, 0)
  soff = pl.multiple_of(s * STRIPE, 256)
  for q in range(8):
    pltpu.sync_copy(zbuf, shared.at[pl.ds(soff + q * ZCH, ZCH)])
  plsc.subcore_barrier()

  fetches = [None, None]

  def _fetch(b):
    return (pltpu.async_copy(src_hbm.at[b, s], srcv, sem1),
            pltpu.async_copy(dst_hbm.at[b, s], dstv, sem2))

  b0 = c * GPC
  fetches = _fetch(b0)
  for r in range(GPC):
    b = b0 + r
    fetches[0].wait()
    fetches[1].wait()
    adds = []
    for j in range(ROWS):
      for k in range(8):
        sl = pl.ds(k * 16, 16)
        sv = srcv[j, sl]
        dv = dstv[j, sl]
        idxb[j, sl] = (lax.shift_right_logical(sv, 7) * CBSZ
                       + dv * 128 + lax.bitwise_and(sv, 127))
      adds.append(pltpu.async_copy(valb, shared.at[idxb.at[j]],
                                   sem3, add=True))
    for cp in adds:
      cp.wait()
    plsc.subcore_barrier()
    obase = pl.multiple_of(b * NPNP, 256)
    out_cp = pltpu.async_copy(shared.at[pl.ds(soff, STRIPE)],
                              out_hbm.at[pl.ds(obase + soff, STRIPE)], sem1)
    out_cp.wait()
    plsc.subcore_barrier()
    if r + 1 < GPC:
      fetches = _fetch(b + 1)
      subs = [pltpu.async_copy(nvalb, shared.at[idxb.at[j]],
                               sem3, add=True) for j in range(ROWS)]
      for cp in subs:
        cp.wait()


def _build_adjacency(dst4, src4):
  mesh = plsc.VectorSubcoreMesh(core_axis_name="c", subcore_axis_name="s")
  f = pl.kernel(
      _sc_body,
      out_type=jax.ShapeDtypeStruct((B * NPNP,), jnp.float32),
      mesh=mesh,
      scratch_types=[
          pltpu.VMEM_SHARED((NPNP + 256,), jnp.float32),
          pltpu.VMEM((ROWS, 128), jnp.int32),
          pltpu.VMEM((ROWS, 128), jnp.int32),
          pltpu.VMEM((ROWS, 128), jnp.int32),
          pltpu.VMEM((128,), jnp.float32),
          pltpu.VMEM((128,), jnp.float32),
          pltpu.VMEM((ZCH,), jnp.float32),
          pltpu.SemaphoreType.DMA,
          pltpu.SemaphoreType.DMA,
          pltpu.SemaphoreType.DMA,
      ],
  )
  return f(dst4, src4)


def _tc_body(a_ref, x_ref, w_ref, b_ref, o_ref, *ans):
  h = pl.program_id(1)
  HCB = CB // HSTEPS

  def _mk_asm(hh):
    def _asm():
      for g in range(TG):
        for cbl in range(HCB):
          cb = hh * HCB + cbl
          ans[g][:, 128 * cb:128 * (cb + 1)] = (
              a_ref[g, cbl].astype(jnp.bfloat16))
    return _asm

  for hh in range(HSTEPS - 1):
    pl.when(h == hh)(_mk_asm(hh))

  @pl.when(h == HSTEPS - 1)
  def _():
    _mk_asm(HSTEPS - 1)()
    ones = jnp.ones((Np, 1), jnp.bfloat16)
    dis = []
    for g in range(TG):
      deg = jnp.dot(ans[g][...], ones,
                    preferred_element_type=jnp.float32) + 1.0
      dis.append(lax.rsqrt(deg))
    xs = [x_ref[g] for g in range(TG)]
    for l in range(L):
      w = w_ref[l]
      bias = b_ref[l][None, :]
      hs = [None] * TG
      for g in range(TG):
        hh2 = jnp.dot(xs[g], w, preferred_element_type=jnp.float32)
        hs[g] = hh2 * dis[g]
      for g in range(TG):
        y = jnp.dot(ans[g][...], hs[g].astype(jnp.bfloat16),
                    preferred_element_type=jnp.float32) + hs[g]
        xs[g] = jnp.tanh(y * dis[g] + bias)
    for g in range(TG):
      o_ref[g] = xs[g]


def _gcn_stack(adj, x_pad, wst, bst):
  return pl.pallas_call(
      _tc_body,
      grid=(B // TG, HSTEPS),
      in_specs=[
          pl.BlockSpec((TG, CB // HSTEPS, Np, 128), lambda b, h: (b, h, 0, 0)),
          pl.BlockSpec((TG, Np, D), lambda b, h: (b, 0, 0)),
          pl.BlockSpec((L, D, D), lambda b, h: (0, 0, 0)),
          pl.BlockSpec((L, D), lambda b, h: (0, 0)),
      ],
      out_specs=pl.BlockSpec((TG, Np, D), lambda b, h: (b, 0, 0)),
      out_shape=jax.ShapeDtypeStruct((B, N, D), jnp.float32),
      scratch_shapes=[pltpu.VMEM((Np, Np), jnp.bfloat16)
                      for _ in range(TG)],
  )(adj, x_pad, wst, bst)


@jax.jit
def kernel(batch_node_tsr, edge_tsr_list, batch_last_node_idx_list,
           W0, b0, W1, b1, W2, b2, W3, b3):
  del batch_last_node_idx_list
  src = edge_tsr_list[:, 0, :].reshape(B, NS, EP)
  dst = edge_tsr_list[:, 1, :].reshape(B, NS, EP)
  pad = ((0, 0), (0, 0), (0, EPP - EP))
  src4 = jnp.pad(src, pad, constant_values=Np).reshape(B, NS, ROWS, 128)
  dst4 = jnp.pad(dst, pad).reshape(B, NS, ROWS, 128)
  adj = _build_adjacency(dst4, src4).reshape(B, CB, Np, 128)

  x_pad = jnp.pad(batch_node_tsr, ((0, 0), (0, Np - N), (0, 0)))
  wst = jnp.stack([W0, W1, W2, W3])
  bst = jnp.stack([b0, b1, b2, b3])
  return _gcn_stack(adj, x_pad, wst, bst)
